# baseline (speedup 1.0000x reference)
import functools

import jax
import jax.numpy as jnp
from jax import lax
from jax.experimental import pallas as pl
from jax.experimental.pallas import tpu as pltpu
from jax.experimental.pallas import tpu_sc as plsc

BATCH = 16384
N_ACTIONS = 64
NW = 32
BPW = BATCH // NW
LANES = 16
CHUNKS = BPW // LANES


def _run(s_hbm, a_hbm, t_hbm, out_hbm, s_v, a_v, gr_v, out_v, sem):
    wid = lax.axis_index("s") * 2 + lax.axis_index("c")
    base = wid * BPW

    stage_s = pltpu.async_copy(s_hbm.at[pl.ds(base, BPW)], s_v, sem)
    stage_a = pltpu.async_copy(a_hbm.at[pl.ds(base, BPW)], a_v, sem)
    stage_s.wait()
    stage_a.wait()

    @plsc.parallel_loop(0, CHUNKS, unroll=1)
    def fetch_chunk(c):
        sv = s_v[pl.ds(c * LANES, LANES)]
        av = a_v[pl.ds(c * LANES, LANES)]
        gv = sv // LANES
        for k in range(LANES):
            pltpu.async_copy(
                t_hbm.at[av[k], pl.ds(gv[k] * LANES, LANES)],
                gr_v.at[pl.ds((c * LANES + k) * LANES, LANES)], sem)

    pltpu.make_async_copy(
        out_hbm.at[pl.ds(0, BPW * LANES)], gr_v, sem).wait()

    def pick_chunk(c, carry):
        sv = s_v[pl.ds(c * LANES, LANES)]
        flat = (lax.iota(jnp.int32, LANES) + c * LANES) * LANES + sv % LANES
        out_v[pl.ds(c * LANES, LANES)] = plsc.load_gather(gr_v, [flat])
        return carry

    lax.fori_loop(0, CHUNKS, pick_chunk, 0)

    pltpu.sync_copy(out_v, out_hbm.at[pl.ds(base, BPW)])


def kernel(state, action, table):
    s = state.astype(jnp.int32)
    a = action.astype(jnp.int32)
    t = table.T

    mesh = plsc.VectorSubcoreMesh(core_axis_name="c", subcore_axis_name="s")
    run = functools.partial(
        pl.kernel,
        mesh=mesh,
        compiler_params=pltpu.CompilerParams(needs_layout_passes=False),
        out_type=jax.ShapeDtypeStruct((BATCH,), jnp.float32),
        scratch_types=[
            pltpu.VMEM((BPW,), jnp.int32),
            pltpu.VMEM((BPW,), jnp.int32),
            pltpu.VMEM((BPW * LANES,), jnp.float32),
            pltpu.VMEM((BPW,), jnp.float32),
            pltpu.SemaphoreType.DMA,
        ],
    )(_run)
    return run(s, a, t)


# speedup vs baseline: 1.0439x; 1.0439x over previous
import functools

import jax
import jax.numpy as jnp
from jax import lax
from jax.experimental import pallas as pl
from jax.experimental.pallas import tpu as pltpu
from jax.experimental.pallas import tpu_sc as plsc

BATCH = 16384
N_ACTIONS = 64
NW = 32
BPW = BATCH // NW
LANES = 16
CHUNKS = BPW // LANES


def _run(s_hbm, a_hbm, t_hbm, out_hbm, s_v, a_v, gr_v, out_v, sem):
    wid = lax.axis_index("s") * 2 + lax.axis_index("c")
    base = wid * BPW

    stage_s = pltpu.async_copy(s_hbm.at[pl.ds(base, BPW)], s_v, sem)
    stage_a = pltpu.async_copy(a_hbm.at[pl.ds(base, BPW)], a_v, sem)
    stage_s.wait()
    stage_a.wait()

    def fetch_chunk(c2, carry):
        for h in range(2):
            c = c2 * 2 + h
            sv = s_v[pl.ds(c * LANES, LANES)]
            av = a_v[pl.ds(c * LANES, LANES)]
            gv = sv // LANES
            for k in range(LANES):
                pltpu.async_copy(
                    t_hbm.at[av[k], pl.ds(gv[k] * LANES, LANES)],
                    gr_v.at[pl.ds((c * LANES + k) * LANES, LANES)], sem)
        return carry

    lax.fori_loop(0, CHUNKS // 2, fetch_chunk, 0)

    pltpu.make_async_copy(
        out_hbm.at[pl.ds(0, BPW * LANES)], gr_v, sem).wait()

    def pick_chunk(c, carry):
        sv = s_v[pl.ds(c * LANES, LANES)]
        flat = (lax.iota(jnp.int32, LANES) + c * LANES) * LANES + sv % LANES
        out_v[pl.ds(c * LANES, LANES)] = plsc.load_gather(gr_v, [flat])
        return carry

    lax.fori_loop(0, CHUNKS, pick_chunk, 0)

    pltpu.sync_copy(out_v, out_hbm.at[pl.ds(base, BPW)])


def kernel(state, action, table):
    s = state.astype(jnp.int32)
    a = action.astype(jnp.int32)
    t = table.T

    mesh = plsc.VectorSubcoreMesh(core_axis_name="c", subcore_axis_name="s")
    run = functools.partial(
        pl.kernel,
        mesh=mesh,
        compiler_params=pltpu.CompilerParams(needs_layout_passes=False),
        out_type=jax.ShapeDtypeStruct((BATCH,), jnp.float32),
        scratch_types=[
            pltpu.VMEM((BPW,), jnp.int32),
            pltpu.VMEM((BPW,), jnp.int32),
            pltpu.VMEM((BPW * LANES,), jnp.float32),
            pltpu.VMEM((BPW,), jnp.float32),
            pltpu.SemaphoreType.DMA,
        ],
    )(_run)
    return run(s, a, t)


# speedup vs baseline: 1.0575x; 1.0131x over previous
import functools

import jax
import jax.numpy as jnp
from jax import lax
from jax.experimental import pallas as pl
from jax.experimental.pallas import tpu as pltpu
from jax.experimental.pallas import tpu_sc as plsc

BATCH = 16384
N_ACTIONS = 64
NW = 32
BPW = BATCH // NW
LANES = 16
CHUNKS = BPW // LANES


def _run(s_hbm, a_hbm, t_hbm, out_hbm, s_v, a_v, gr_v, out_v, sem):
    wid = lax.axis_index("s") * 2 + lax.axis_index("c")
    base = wid * BPW

    stage_s = pltpu.async_copy(s_hbm.at[pl.ds(base, BPW)], s_v, sem)
    stage_a = pltpu.async_copy(a_hbm.at[pl.ds(base, BPW)], a_v, sem)
    stage_s.wait()
    stage_a.wait()

    def fetch_chunk(c, carry):
        sv = s_v[pl.ds(c * LANES, LANES)]
        av = a_v[pl.ds(c * LANES, LANES)]
        gv = sv // LANES
        for k in range(LANES):
            pltpu.async_copy(
                t_hbm.at[av[k], pl.ds(gv[k] * LANES, LANES)],
                gr_v.at[pl.ds((c * LANES + k) * LANES, LANES)], sem)
        return carry

    lax.fori_loop(0, CHUNKS, fetch_chunk, 0)

    pltpu.make_async_copy(
        out_hbm.at[pl.ds(0, BPW * LANES)], gr_v, sem).wait()

    def pick_chunk(c, carry):
        sv = s_v[pl.ds(c * LANES, LANES)]
        flat = (lax.iota(jnp.int32, LANES) + c * LANES) * LANES + sv % LANES
        out_v[pl.ds(c * LANES, LANES)] = plsc.load_gather(gr_v, [flat])
        return carry

    lax.fori_loop(0, CHUNKS, pick_chunk, 0)

    pltpu.sync_copy(out_v, out_hbm.at[pl.ds(base, BPW)])


def kernel(state, action, table):
    s = state.astype(jnp.int32)
    a = action.astype(jnp.int32)
    t = table.T

    mesh = plsc.VectorSubcoreMesh(core_axis_name="c", subcore_axis_name="s")
    run = functools.partial(
        pl.kernel,
        mesh=mesh,
        compiler_params=pltpu.CompilerParams(needs_layout_passes=False),
        out_type=jax.ShapeDtypeStruct((BATCH,), jnp.float32),
        scratch_types=[
            pltpu.VMEM((BPW,), jnp.int32),
            pltpu.VMEM((BPW,), jnp.int32),
            pltpu.VMEM((BPW * LANES,), jnp.float32),
            pltpu.VMEM((BPW,), jnp.float32),
            pltpu.SemaphoreType.DMA,
        ],
    )(_run)
    return run(s, a, t)
